# baseline (speedup 1.0000x reference)
import jax
import jax.numpy as jnp
from jax import lax
from jax.experimental import pallas as pl
from jax.experimental.pallas import tpu as pltpu
from jax.experimental.pallas import tpu_sc as plsc

N_NODES = 10000
N_EDGES = 320000
D_FEAT = 128
D_EDGE = 16
D_OUT = 128

NUM_CORES = 2
NUM_SUBCORES = 16
NW = NUM_CORES * NUM_SUBCORES
E_PER_W = N_EDGES // NW
CHUNK = 40
NITER = E_PER_W // CHUNK
NBUF = 2
D_PACK = D_OUT // 2


def _pack_words(p):
    u = jax.lax.bitcast_convert_type(p.astype(jnp.bfloat16), jnp.uint16)
    lo = u[:, :D_PACK].astype(jnp.int32)
    hi = u[:, D_PACK:].astype(jnp.int32)
    return lo | (hi << 16)


def _proj_body(x_ref, w1_ref, w3_ref, p1_ref, p3_ref):
    xb = x_ref[...]
    p1_ref[...] = _pack_words(
        jnp.dot(xb, w1_ref[...], preferred_element_type=jnp.float32))
    p3_ref[...] = _pack_words(
        jnp.dot(xb, w3_ref[...], preferred_element_type=jnp.float32))


def _final_body(g_ref, a_ref, w2_ref, b_ref, o_ref):
    gw = g_ref[...]
    lo = jax.lax.bitcast_convert_type(gw << 16, jnp.float32)
    hi = jax.lax.bitcast_convert_type(gw & jnp.int32(-65536), jnp.float32)
    a3 = a_ref[...]
    bb = b_ref[...]
    q0 = jnp.dot(a3[:, 0, :], w2_ref[...],
                 preferred_element_type=jnp.float32) + bb
    q1 = jnp.dot(a3[:, 1, :], w2_ref[...],
                 preferred_element_type=jnp.float32) + bb
    o_ref[:, 0, :] = jnp.concatenate(
        [lo[:, :D_PACK], hi[:, :D_PACK]], axis=1) + q0
    o_ref[:, 1, :] = jnp.concatenate(
        [lo[:, D_PACK:], hi[:, D_PACK:]], axis=1) + q1


def _gather_add_body(p1_hbm, p3_hbm, src_hbm, dst_hbm, g_hbm,
                     idx_s, idx_d, rows_s, rows_d, rows_g,
                     sem_g0, sem_g1, sem_o0, sem_o1):
    sems_g = (sem_g0, sem_g1)
    sems_o = (sem_o0, sem_o1)
    wid = lax.axis_index("s") * NUM_CORES + lax.axis_index("c")

    pltpu.sync_copy(src_hbm.at[wid], idx_s)
    pltpu.sync_copy(dst_hbm.at[wid], idx_d)

    def issue_gathers(i, b):
        off = i * CHUNK
        pltpu.async_copy(
            p1_hbm.at[idx_s.at[pl.ds(off, CHUNK)]], rows_s.at[b], sems_g[b])
        pltpu.async_copy(
            p3_hbm.at[idx_d.at[pl.ds(off, CHUNK)]], rows_d.at[b], sems_g[b])

    def wait_gathers(i, b):
        off = i * CHUNK
        pltpu.make_async_copy(
            p1_hbm.at[idx_s.at[pl.ds(off, CHUNK)]], rows_s.at[b], sems_g[b]).wait()
        pltpu.make_async_copy(
            p3_hbm.at[idx_d.at[pl.ds(off, CHUNK)]], rows_d.at[b], sems_g[b]).wait()

    def out_slice(i):
        return g_hbm.at[pl.ds((wid * E_PER_W + i * CHUNK) // 2, CHUNK // 2)]

    for b in range(NBUF):
        issue_gathers(jnp.int32(b), b)

    @pl.loop(0, NITER, step=NBUF)
    def _outer(i0):
        for b in range(NBUF):
            i = i0 + b
            wait_gathers(i, b)

            @pl.when(i >= NBUF)
            def _():
                pltpu.make_async_copy(rows_g.at[b], out_slice(i - NBUF),
                                      sems_o[b]).wait()

            @plsc.parallel_loop(0, CHUNK // 2, unroll=2)
            def _add(r2):
                for half in range(2):
                    r = r2 * 2 + half
                    for c in range(D_PACK // 16):
                        sl = pl.ds(c * 16, 16)
                        s = plsc.bitcast(rows_s[b, r, sl], jnp.bfloat16)
                        d = plsc.bitcast(rows_d[b, r, sl], jnp.bfloat16)
                        rows_g[b, r2, pl.ds(half * D_PACK + c * 16, 16)] = (
                            plsc.bitcast(s + d, jnp.int32))

            pltpu.async_copy(rows_g.at[b], out_slice(i), sems_o[b])

            @pl.when(i + NBUF < NITER)
            def _():
                issue_gathers(i + NBUF, b)

    for b in range(NBUF):
        i = NITER - NBUF + b
        pltpu.make_async_copy(rows_g.at[b], out_slice(jnp.int32(i)),
                              sems_o[b]).wait()


@jax.jit
def kernel(x, edge_attr, edge_index, W, b):
    W1 = W[:D_FEAT]
    W2 = W[D_FEAT:D_FEAT + D_EDGE]
    W3 = W[D_FEAT + D_EDGE:]

    BN = 2000
    P1, P3 = pl.pallas_call(
        _proj_body,
        grid=(N_NODES // BN,),
        in_specs=[
            pl.BlockSpec((BN, D_FEAT), lambda i: (i, 0)),
            pl.BlockSpec((D_FEAT, D_OUT), lambda i: (0, 0)),
            pl.BlockSpec((D_FEAT, D_OUT), lambda i: (0, 0)),
        ],
        out_specs=[
            pl.BlockSpec((BN, D_PACK), lambda i: (i, 0)),
            pl.BlockSpec((BN, D_PACK), lambda i: (i, 0)),
        ],
        out_shape=[
            jax.ShapeDtypeStruct((N_NODES, D_PACK), jnp.int32),
            jax.ShapeDtypeStruct((N_NODES, D_PACK), jnp.int32),
        ],
    )(x, W1, W3)

    src = edge_index[0].reshape(NW, E_PER_W)
    dst = edge_index[1].reshape(NW, E_PER_W)
    mesh = plsc.VectorSubcoreMesh(core_axis_name="c", subcore_axis_name="s")
    Gw = pl.kernel(
        _gather_add_body,
        out_type=jax.ShapeDtypeStruct((N_EDGES // 2, D_OUT), jnp.int32),
        mesh=mesh,
        compiler_params=pltpu.CompilerParams(
            use_tc_tiling_on_sc=False, needs_layout_passes=False),
        scratch_types=[
            pltpu.VMEM((E_PER_W,), jnp.int32),
            pltpu.VMEM((E_PER_W,), jnp.int32),
            pltpu.VMEM((NBUF, CHUNK, D_PACK), jnp.int32),
            pltpu.VMEM((NBUF, CHUNK, D_PACK), jnp.int32),
            pltpu.VMEM((NBUF, CHUNK // 2, D_OUT), jnp.int32),
            pltpu.SemaphoreType.DMA,
            pltpu.SemaphoreType.DMA,
            pltpu.SemaphoreType.DMA,
            pltpu.SemaphoreType.DMA,
        ],
    )(P1, P3, src, dst)

    BE = 4000
    BEH = BE // 2
    b2 = b.reshape(1, D_OUT)
    attr3 = edge_attr.reshape(N_EDGES // 2, 2, D_EDGE)
    out3 = pl.pallas_call(
        _final_body,
        grid=(N_EDGES // BE,),
        in_specs=[
            pl.BlockSpec((BEH, D_OUT), lambda i: (i, 0)),
            pl.BlockSpec((BEH, 2, D_EDGE), lambda i: (i, 0, 0)),
            pl.BlockSpec((D_EDGE, D_OUT), lambda i: (0, 0)),
            pl.BlockSpec((1, D_OUT), lambda i: (0, 0)),
        ],
        out_specs=pl.BlockSpec((BEH, 2, D_OUT), lambda i: (i, 0, 0)),
        out_shape=jax.ShapeDtypeStruct((N_EDGES // 2, 2, D_OUT), jnp.float32),
    )(Gw, attr3, W2, b2)
    return out3.reshape(N_EDGES, D_OUT)


# speedup vs baseline: 1.0542x; 1.0542x over previous
import jax
import jax.numpy as jnp
from jax import lax
from jax.experimental import pallas as pl
from jax.experimental.pallas import tpu as pltpu
from jax.experimental.pallas import tpu_sc as plsc

N_NODES = 10000
N_EDGES = 320000
D_FEAT = 128
D_EDGE = 16
D_OUT = 128

NUM_CORES = 2
NUM_SUBCORES = 16
NW = NUM_CORES * NUM_SUBCORES
E_PER_W = N_EDGES // NW
CHUNK = 40
NITER = E_PER_W // CHUNK
NBUF = 2
NSLOT = 4
D_PACK = D_OUT // 2


def _pack_words(p):
    u = jax.lax.bitcast_convert_type(p.astype(jnp.bfloat16), jnp.uint16)
    lo = u[:, :D_PACK].astype(jnp.int32)
    hi = u[:, D_PACK:].astype(jnp.int32)
    return lo | (hi << 16)


def _proj_body(x_ref, w1_ref, w3_ref, p1_ref, p3_ref):
    xb = x_ref[...]
    p1_ref[...] = _pack_words(
        jnp.dot(xb, w1_ref[...], preferred_element_type=jnp.float32))
    p3_ref[...] = _pack_words(
        jnp.dot(xb, w3_ref[...], preferred_element_type=jnp.float32))


def _final_body(g_ref, a_ref, w2_ref, b_ref, o_ref):
    gw = g_ref[...]
    lo = jax.lax.bitcast_convert_type(gw << 16, jnp.float32)
    hi = jax.lax.bitcast_convert_type(gw & jnp.int32(-65536), jnp.float32)
    a3 = a_ref[...]
    bb = b_ref[...]
    q0 = jnp.dot(a3[:, 0, :], w2_ref[...],
                 preferred_element_type=jnp.float32) + bb
    q1 = jnp.dot(a3[:, 1, :], w2_ref[...],
                 preferred_element_type=jnp.float32) + bb
    o_ref[:, 0, :] = jnp.concatenate(
        [lo[:, :D_PACK], hi[:, :D_PACK]], axis=1) + q0
    o_ref[:, 1, :] = jnp.concatenate(
        [lo[:, D_PACK:], hi[:, D_PACK:]], axis=1) + q1


def _gather_add_body(p1_hbm, p3_hbm, src_hbm, dst_hbm, g_hbm,
                     idx_s, idx_d, rows_s, rows_d, rows_g,
                     sem_g0, sem_g1, sem_o0, sem_o1):
    sems_g = (sem_g0, sem_g1)
    sems_o = (sem_o0, sem_o1)
    wid = lax.axis_index("s") * NUM_CORES + lax.axis_index("c")

    pltpu.sync_copy(src_hbm.at[wid], idx_s)
    pltpu.sync_copy(dst_hbm.at[wid], idx_d)

    def issue_gathers(i, b):
        off = i * CHUNK
        slot = lax.rem(i, NSLOT)
        pltpu.async_copy(
            p1_hbm.at[idx_s.at[pl.ds(off, CHUNK)]], rows_s.at[slot], sems_g[b])
        pltpu.async_copy(
            p3_hbm.at[idx_d.at[pl.ds(off, CHUNK)]], rows_d.at[slot], sems_g[b])

    def wait_gathers(i, b):
        off = i * CHUNK
        slot = lax.rem(i, NSLOT)
        pltpu.make_async_copy(
            p1_hbm.at[idx_s.at[pl.ds(off, CHUNK)]], rows_s.at[slot],
            sems_g[b]).wait()
        pltpu.make_async_copy(
            p3_hbm.at[idx_d.at[pl.ds(off, CHUNK)]], rows_d.at[slot],
            sems_g[b]).wait()

    def out_slice(i):
        return g_hbm.at[pl.ds((wid * E_PER_W + i * CHUNK) // 2, CHUNK // 2)]

    for b in range(NBUF):
        issue_gathers(jnp.int32(b), b)

    @pl.loop(0, NITER, step=NBUF)
    def _outer(i0):
        for b in range(NBUF):
            i = i0 + b
            wait_gathers(i, b)

            @pl.when(i + NBUF < NITER)
            def _():
                issue_gathers(i + NBUF, b)

            @pl.when(i >= NBUF)
            def _():
                pltpu.make_async_copy(rows_g.at[b], out_slice(i - NBUF),
                                      sems_o[b]).wait()

            slot = lax.rem(i, NSLOT)

            @plsc.parallel_loop(0, CHUNK // 2, unroll=2)
            def _add(r2):
                for half in range(2):
                    r = r2 * 2 + half
                    for c in range(D_PACK // 16):
                        sl = pl.ds(c * 16, 16)
                        s = plsc.bitcast(rows_s[slot, r, sl], jnp.bfloat16)
                        d = plsc.bitcast(rows_d[slot, r, sl], jnp.bfloat16)
                        rows_g[b, r2, pl.ds(half * D_PACK + c * 16, 16)] = (
                            plsc.bitcast(s + d, jnp.int32))

            pltpu.async_copy(rows_g.at[b], out_slice(i), sems_o[b])

    for b in range(NBUF):
        i = NITER - NBUF + b
        pltpu.make_async_copy(rows_g.at[b], out_slice(jnp.int32(i)),
                              sems_o[b]).wait()


@jax.jit
def kernel(x, edge_attr, edge_index, W, b):
    W1 = W[:D_FEAT]
    W2 = W[D_FEAT:D_FEAT + D_EDGE]
    W3 = W[D_FEAT + D_EDGE:]

    BN = 2000
    P1, P3 = pl.pallas_call(
        _proj_body,
        grid=(N_NODES // BN,),
        in_specs=[
            pl.BlockSpec((BN, D_FEAT), lambda i: (i, 0)),
            pl.BlockSpec((D_FEAT, D_OUT), lambda i: (0, 0)),
            pl.BlockSpec((D_FEAT, D_OUT), lambda i: (0, 0)),
        ],
        out_specs=[
            pl.BlockSpec((BN, D_PACK), lambda i: (i, 0)),
            pl.BlockSpec((BN, D_PACK), lambda i: (i, 0)),
        ],
        out_shape=[
            jax.ShapeDtypeStruct((N_NODES, D_PACK), jnp.int32),
            jax.ShapeDtypeStruct((N_NODES, D_PACK), jnp.int32),
        ],
    )(x, W1, W3)

    src = edge_index[0].reshape(NW, E_PER_W)
    dst = edge_index[1].reshape(NW, E_PER_W)
    mesh = plsc.VectorSubcoreMesh(core_axis_name="c", subcore_axis_name="s")
    Gw = pl.kernel(
        _gather_add_body,
        out_type=jax.ShapeDtypeStruct((N_EDGES // 2, D_OUT), jnp.int32),
        mesh=mesh,
        compiler_params=pltpu.CompilerParams(
            use_tc_tiling_on_sc=False, needs_layout_passes=False),
        scratch_types=[
            pltpu.VMEM((E_PER_W,), jnp.int32),
            pltpu.VMEM((E_PER_W,), jnp.int32),
            pltpu.VMEM((NSLOT, CHUNK, D_PACK), jnp.int32),
            pltpu.VMEM((NSLOT, CHUNK, D_PACK), jnp.int32),
            pltpu.VMEM((NBUF, CHUNK // 2, D_OUT), jnp.int32),
            pltpu.SemaphoreType.DMA,
            pltpu.SemaphoreType.DMA,
            pltpu.SemaphoreType.DMA,
            pltpu.SemaphoreType.DMA,
        ],
    )(P1, P3, src, dst)

    BE = 4000
    BEH = BE // 2
    b2 = b.reshape(1, D_OUT)
    attr3 = edge_attr.reshape(N_EDGES // 2, 2, D_EDGE)
    out3 = pl.pallas_call(
        _final_body,
        grid=(N_EDGES // BE,),
        in_specs=[
            pl.BlockSpec((BEH, D_OUT), lambda i: (i, 0)),
            pl.BlockSpec((BEH, 2, D_EDGE), lambda i: (i, 0, 0)),
            pl.BlockSpec((D_EDGE, D_OUT), lambda i: (0, 0)),
            pl.BlockSpec((1, D_OUT), lambda i: (0, 0)),
        ],
        out_specs=pl.BlockSpec((BEH, 2, D_OUT), lambda i: (i, 0, 0)),
        out_shape=jax.ShapeDtypeStruct((N_EDGES // 2, 2, D_OUT), jnp.float32),
    )(Gw, attr3, W2, b2)
    return out3.reshape(N_EDGES, D_OUT)


# speedup vs baseline: 1.0621x; 1.0075x over previous
import jax
import jax.numpy as jnp
from jax import lax
from jax.experimental import pallas as pl
from jax.experimental.pallas import tpu as pltpu
from jax.experimental.pallas import tpu_sc as plsc

N_NODES = 10000
N_EDGES = 320000
D_FEAT = 128
D_EDGE = 16
D_OUT = 128

NUM_CORES = 2
NUM_SUBCORES = 16
NW = NUM_CORES * NUM_SUBCORES
E_PER_W = N_EDGES // NW
CHUNK = 40
NITER = E_PER_W // CHUNK
NBUF = 2
NSLOT = 4
D_PACK = D_OUT // 2


def _pack_words(p):
    u = jax.lax.bitcast_convert_type(p.astype(jnp.bfloat16), jnp.uint16)
    lo = u[:, :D_PACK].astype(jnp.int32)
    hi = u[:, D_PACK:].astype(jnp.int32)
    return lo | (hi << 16)


def _proj_body(x_ref, w1_ref, w3_ref, p1_ref, p3_ref):
    xb = x_ref[...]
    p1_ref[...] = _pack_words(
        jnp.dot(xb, w1_ref[...], preferred_element_type=jnp.float32))
    p3_ref[...] = _pack_words(
        jnp.dot(xb, w3_ref[...], preferred_element_type=jnp.float32))


def _final_body(g_ref, a_ref, w2_ref, b_ref, o_ref):
    gw = g_ref[...]
    lo = jax.lax.bitcast_convert_type(gw << 16, jnp.float32)
    hi = jax.lax.bitcast_convert_type(gw & jnp.int32(-65536), jnp.float32)
    a3 = a_ref[...]
    bb = b_ref[...]
    q0 = jnp.dot(a3[:, 0, :], w2_ref[...],
                 preferred_element_type=jnp.float32) + bb
    q1 = jnp.dot(a3[:, 1, :], w2_ref[...],
                 preferred_element_type=jnp.float32) + bb
    o_ref[:, 0, :] = jnp.concatenate(
        [lo[:, :D_PACK], hi[:, :D_PACK]], axis=1) + q0
    o_ref[:, 1, :] = jnp.concatenate(
        [lo[:, D_PACK:], hi[:, D_PACK:]], axis=1) + q1


def _gather_add_body(p1_hbm, p3_hbm, src_hbm, dst_hbm, g_hbm,
                     idx_s, idx_d, rows_s, rows_d, rows_g,
                     sem_g0, sem_g1, sem_o0, sem_o1):
    sems_g = (sem_g0, sem_g1)
    sems_o = (sem_o0, sem_o1)
    wid = lax.axis_index("s") * NUM_CORES + lax.axis_index("c")

    pltpu.sync_copy(src_hbm.at[wid], idx_s)
    pltpu.sync_copy(dst_hbm.at[wid], idx_d)

    def issue_gathers(i, b):
        off = i * CHUNK
        slot = lax.rem(i, NSLOT)
        pltpu.async_copy(
            p1_hbm.at[idx_s.at[pl.ds(off, CHUNK)]], rows_s.at[slot], sems_g[b])
        pltpu.async_copy(
            p3_hbm.at[idx_d.at[pl.ds(off, CHUNK)]], rows_d.at[slot], sems_g[b])

    def wait_gathers(i, b):
        off = i * CHUNK
        slot = lax.rem(i, NSLOT)
        pltpu.make_async_copy(
            p1_hbm.at[idx_s.at[pl.ds(off, CHUNK)]], rows_s.at[slot],
            sems_g[b]).wait()
        pltpu.make_async_copy(
            p3_hbm.at[idx_d.at[pl.ds(off, CHUNK)]], rows_d.at[slot],
            sems_g[b]).wait()

    def out_slice(i):
        return g_hbm.at[pl.ds((wid * E_PER_W + i * CHUNK) // 2, CHUNK // 2)]

    for b in range(NBUF):
        issue_gathers(jnp.int32(b), b)

    @pl.loop(0, NITER, step=NBUF)
    def _outer(i0):
        for b in range(NBUF):
            i = i0 + b
            wait_gathers(i, b)

            @pl.when(i + NBUF < NITER)
            def _():
                issue_gathers(i + NBUF, b)

            @pl.when(i >= NBUF)
            def _():
                pltpu.make_async_copy(rows_g.at[b], out_slice(i - NBUF),
                                      sems_o[b]).wait()

            slot = lax.rem(i, NSLOT)

            @plsc.parallel_loop(0, CHUNK // 2, unroll=2)
            def _add(r2):
                for half in range(2):
                    r = r2 * 2 + half
                    for c in range(D_PACK // 16):
                        sl = pl.ds(c * 16, 16)
                        s = plsc.bitcast(rows_s[slot, r, sl], jnp.bfloat16)
                        d = plsc.bitcast(rows_d[slot, r, sl], jnp.bfloat16)
                        rows_g[b, r2, pl.ds(half * D_PACK + c * 16, 16)] = (
                            plsc.bitcast(s + d, jnp.int32))

            pltpu.async_copy(rows_g.at[b], out_slice(i), sems_o[b])

    for b in range(NBUF):
        i = NITER - NBUF + b
        pltpu.make_async_copy(rows_g.at[b], out_slice(jnp.int32(i)),
                              sems_o[b]).wait()


@jax.jit
def kernel(x, edge_attr, edge_index, W, b):
    W1 = W[:D_FEAT]
    W2 = W[D_FEAT:D_FEAT + D_EDGE]
    W3 = W[D_FEAT + D_EDGE:]

    BN = 2000
    P1, P3 = pl.pallas_call(
        _proj_body,
        grid=(N_NODES // BN,),
        in_specs=[
            pl.BlockSpec((BN, D_FEAT), lambda i: (i, 0)),
            pl.BlockSpec((D_FEAT, D_OUT), lambda i: (0, 0)),
            pl.BlockSpec((D_FEAT, D_OUT), lambda i: (0, 0)),
        ],
        out_specs=[
            pl.BlockSpec((BN, D_PACK), lambda i: (i, 0)),
            pl.BlockSpec((BN, D_PACK), lambda i: (i, 0)),
        ],
        out_shape=[
            jax.ShapeDtypeStruct((N_NODES, D_PACK), jnp.int32),
            jax.ShapeDtypeStruct((N_NODES, D_PACK), jnp.int32),
        ],
    )(x, W1, W3)

    src = edge_index[0].reshape(NW, E_PER_W)
    dst = edge_index[1].reshape(NW, E_PER_W)
    mesh = plsc.VectorSubcoreMesh(core_axis_name="c", subcore_axis_name="s")
    Gw = pl.kernel(
        _gather_add_body,
        out_type=jax.ShapeDtypeStruct((N_EDGES // 2, D_OUT), jnp.int32),
        mesh=mesh,
        compiler_params=pltpu.CompilerParams(
            use_tc_tiling_on_sc=False, needs_layout_passes=False),
        scratch_types=[
            pltpu.VMEM((E_PER_W,), jnp.int32),
            pltpu.VMEM((E_PER_W,), jnp.int32),
            pltpu.VMEM((NSLOT, CHUNK, D_PACK), jnp.int32),
            pltpu.VMEM((NSLOT, CHUNK, D_PACK), jnp.int32),
            pltpu.VMEM((NBUF, CHUNK // 2, D_OUT), jnp.int32),
            pltpu.SemaphoreType.DMA,
            pltpu.SemaphoreType.DMA,
            pltpu.SemaphoreType.DMA,
            pltpu.SemaphoreType.DMA,
        ],
    )(P1, P3, src, dst)

    BE = 8000
    BEH = BE // 2
    b2 = b.reshape(1, D_OUT)
    attr3 = edge_attr.reshape(N_EDGES // 2, 2, D_EDGE)
    out3 = pl.pallas_call(
        _final_body,
        grid=(N_EDGES // BE,),
        in_specs=[
            pl.BlockSpec((BEH, D_OUT), lambda i: (i, 0)),
            pl.BlockSpec((BEH, 2, D_EDGE), lambda i: (i, 0, 0)),
            pl.BlockSpec((D_EDGE, D_OUT), lambda i: (0, 0)),
            pl.BlockSpec((1, D_OUT), lambda i: (0, 0)),
        ],
        out_specs=pl.BlockSpec((BEH, 2, D_OUT), lambda i: (i, 0, 0)),
        out_shape=jax.ShapeDtypeStruct((N_EDGES // 2, 2, D_OUT), jnp.float32),
    )(Gw, attr3, W2, b2)
    return out3.reshape(N_EDGES, D_OUT)


# speedup vs baseline: 1.1561x; 1.0885x over previous
import jax
import jax.numpy as jnp
from jax import lax
from jax.experimental import pallas as pl
from jax.experimental.pallas import tpu as pltpu
from jax.experimental.pallas import tpu_sc as plsc

N_NODES = 10000
N_EDGES = 320000
D_FEAT = 128
D_EDGE = 16
D_OUT = 128

NUM_CORES = 2
NUM_SUBCORES = 16
NW = NUM_CORES * NUM_SUBCORES
E_PER_W = N_EDGES // NW
CHUNK = 80
NITER = E_PER_W // CHUNK
NBUF = 2
NSLOT = 4
D_PACK = D_OUT // 2


def _pack_words(p):
    u = jax.lax.bitcast_convert_type(p.astype(jnp.bfloat16), jnp.uint16)
    lo = u[:, :D_PACK].astype(jnp.int32)
    hi = u[:, D_PACK:].astype(jnp.int32)
    return lo | (hi << 16)


def _proj_body(x_ref, w1_ref, w3_ref, p1_ref, p3_ref):
    xb = x_ref[...]
    p1_ref[...] = _pack_words(
        jnp.dot(xb, w1_ref[...], preferred_element_type=jnp.float32))
    p3_ref[...] = _pack_words(
        jnp.dot(xb, w3_ref[...], preferred_element_type=jnp.float32))


def _final_body(g_ref, a_ref, w2_ref, b_ref, o_ref):
    gw = g_ref[...]
    lo = jax.lax.bitcast_convert_type(gw << 16, jnp.float32)
    hi = jax.lax.bitcast_convert_type(gw & jnp.int32(-65536), jnp.float32)
    a3 = a_ref[...]
    bb = b_ref[...]
    q0 = jnp.dot(a3[:, 0, :], w2_ref[...],
                 preferred_element_type=jnp.float32) + bb
    q1 = jnp.dot(a3[:, 1, :], w2_ref[...],
                 preferred_element_type=jnp.float32) + bb
    o_ref[:, 0, :] = jnp.concatenate(
        [lo[:, :D_PACK], hi[:, :D_PACK]], axis=1) + q0
    o_ref[:, 1, :] = jnp.concatenate(
        [lo[:, D_PACK:], hi[:, D_PACK:]], axis=1) + q1


def _gather_add_body(p1_hbm, p3_hbm, src_hbm, dst_hbm, g_hbm,
                     idx_s, idx_d, rows_s, rows_d, rows_g,
                     sem_g0, sem_g1, sem_o0, sem_o1):
    sems_g = (sem_g0, sem_g1)
    sems_o = (sem_o0, sem_o1)
    wid = lax.axis_index("s") * NUM_CORES + lax.axis_index("c")

    pltpu.sync_copy(src_hbm.at[wid], idx_s)
    pltpu.sync_copy(dst_hbm.at[wid], idx_d)

    def issue_gathers(i, b):
        off = i * CHUNK
        slot = lax.rem(i, NSLOT)
        pltpu.async_copy(
            p1_hbm.at[idx_s.at[pl.ds(off, CHUNK)]], rows_s.at[slot], sems_g[b])
        pltpu.async_copy(
            p3_hbm.at[idx_d.at[pl.ds(off, CHUNK)]], rows_d.at[slot], sems_g[b])

    def wait_gathers(i, b):
        off = i * CHUNK
        slot = lax.rem(i, NSLOT)
        pltpu.make_async_copy(
            p1_hbm.at[idx_s.at[pl.ds(off, CHUNK)]], rows_s.at[slot],
            sems_g[b]).wait()
        pltpu.make_async_copy(
            p3_hbm.at[idx_d.at[pl.ds(off, CHUNK)]], rows_d.at[slot],
            sems_g[b]).wait()

    def out_slice(i):
        return g_hbm.at[pl.ds((wid * E_PER_W + i * CHUNK) // 2, CHUNK // 2)]

    def process(i, b):
        wait_gathers(i, b)

        @pl.when(i + NBUF < NITER)
        def _():
            issue_gathers(i + NBUF, b)

        @pl.when(i >= NBUF)
        def _():
            pltpu.make_async_copy(rows_g.at[b], out_slice(i - NBUF),
                                  sems_o[b]).wait()

        slot = lax.rem(i, NSLOT)

        @plsc.parallel_loop(0, CHUNK // 2, unroll=2)
        def _add(r2):
            for half in range(2):
                r = r2 * 2 + half
                for c in range(D_PACK // 16):
                    sl = pl.ds(c * 16, 16)
                    s = plsc.bitcast(rows_s[slot, r, sl], jnp.bfloat16)
                    d = plsc.bitcast(rows_d[slot, r, sl], jnp.bfloat16)
                    rows_g[b, r2, pl.ds(half * D_PACK + c * 16, 16)] = (
                        plsc.bitcast(s + d, jnp.int32))

        pltpu.async_copy(rows_g.at[b], out_slice(i), sems_o[b])

    for b in range(NBUF):
        issue_gathers(jnp.int32(b), b)

    NMAIN = (NITER // NBUF) * NBUF

    @pl.loop(0, NMAIN, step=NBUF)
    def _outer(i0):
        for b in range(NBUF):
            process(i0 + b, b)

    for i in range(NMAIN, NITER):
        process(jnp.int32(i), i % NBUF)

    for b in range(NBUF):
        i = NITER - 1 - ((NITER - 1 - b) % NBUF)
        pltpu.make_async_copy(rows_g.at[b], out_slice(jnp.int32(i)),
                              sems_o[b]).wait()


@jax.jit
def kernel(x, edge_attr, edge_index, W, b):
    W1 = W[:D_FEAT]
    W2 = W[D_FEAT:D_FEAT + D_EDGE]
    W3 = W[D_FEAT + D_EDGE:]

    BN = 2000
    P1, P3 = pl.pallas_call(
        _proj_body,
        grid=(N_NODES // BN,),
        in_specs=[
            pl.BlockSpec((BN, D_FEAT), lambda i: (i, 0)),
            pl.BlockSpec((D_FEAT, D_OUT), lambda i: (0, 0)),
            pl.BlockSpec((D_FEAT, D_OUT), lambda i: (0, 0)),
        ],
        out_specs=[
            pl.BlockSpec((BN, D_PACK), lambda i: (i, 0)),
            pl.BlockSpec((BN, D_PACK), lambda i: (i, 0)),
        ],
        out_shape=[
            jax.ShapeDtypeStruct((N_NODES, D_PACK), jnp.int32),
            jax.ShapeDtypeStruct((N_NODES, D_PACK), jnp.int32),
        ],
    )(x, W1, W3)

    src = edge_index[0].reshape(NW, E_PER_W)
    dst = edge_index[1].reshape(NW, E_PER_W)
    mesh = plsc.VectorSubcoreMesh(core_axis_name="c", subcore_axis_name="s")
    Gw = pl.kernel(
        _gather_add_body,
        out_type=jax.ShapeDtypeStruct((N_EDGES // 2, D_OUT), jnp.int32),
        mesh=mesh,
        compiler_params=pltpu.CompilerParams(
            use_tc_tiling_on_sc=False, needs_layout_passes=False),
        scratch_types=[
            pltpu.VMEM((E_PER_W,), jnp.int32),
            pltpu.VMEM((E_PER_W,), jnp.int32),
            pltpu.VMEM((NSLOT, CHUNK, D_PACK), jnp.int32),
            pltpu.VMEM((NSLOT, CHUNK, D_PACK), jnp.int32),
            pltpu.VMEM((NBUF, CHUNK // 2, D_OUT), jnp.int32),
            pltpu.SemaphoreType.DMA,
            pltpu.SemaphoreType.DMA,
            pltpu.SemaphoreType.DMA,
            pltpu.SemaphoreType.DMA,
        ],
    )(P1, P3, src, dst)

    BE = 8000
    BEH = BE // 2
    b2 = b.reshape(1, D_OUT)
    attr3 = edge_attr.reshape(N_EDGES // 2, 2, D_EDGE)
    out3 = pl.pallas_call(
        _final_body,
        grid=(N_EDGES // BE,),
        in_specs=[
            pl.BlockSpec((BEH, D_OUT), lambda i: (i, 0)),
            pl.BlockSpec((BEH, 2, D_EDGE), lambda i: (i, 0, 0)),
            pl.BlockSpec((D_EDGE, D_OUT), lambda i: (0, 0)),
            pl.BlockSpec((1, D_OUT), lambda i: (0, 0)),
        ],
        out_specs=pl.BlockSpec((BEH, 2, D_OUT), lambda i: (i, 0, 0)),
        out_shape=jax.ShapeDtypeStruct((N_EDGES // 2, 2, D_OUT), jnp.float32),
    )(Gw, attr3, W2, b2)
    return out3.reshape(N_EDGES, D_OUT)


# speedup vs baseline: 1.1807x; 1.0213x over previous
import jax
import jax.numpy as jnp
from jax import lax
from jax.experimental import pallas as pl
from jax.experimental.pallas import tpu as pltpu
from jax.experimental.pallas import tpu_sc as plsc

N_NODES = 10000
N_EDGES = 320000
D_FEAT = 128
D_EDGE = 16
D_OUT = 128

NUM_CORES = 2
NUM_SUBCORES = 16
NW = NUM_CORES * NUM_SUBCORES
E_PER_W = N_EDGES // NW
CHUNK = 80
NITER = E_PER_W // CHUNK
NBUF = 2
NSLOT = 4
D_PACK = D_OUT // 2


def _pack_words(p):
    u = jax.lax.bitcast_convert_type(p.astype(jnp.bfloat16), jnp.uint16)
    lo = u[:, :D_PACK].astype(jnp.int32)
    hi = u[:, D_PACK:].astype(jnp.int32)
    return lo | (hi << 16)


def _proj_body(x_ref, w1_ref, w3_ref, p1_ref, p3_ref):
    xb = x_ref[...]
    p1_ref[...] = _pack_words(
        jnp.dot(xb, w1_ref[...], preferred_element_type=jnp.float32))
    p3_ref[...] = _pack_words(
        jnp.dot(xb, w3_ref[...], preferred_element_type=jnp.float32))


def _final_body(g_ref, a_ref, w2_ref, b_ref, o_ref):
    gw = g_ref[...]
    lo = jax.lax.bitcast_convert_type(gw << 16, jnp.float32)
    hi = jax.lax.bitcast_convert_type(gw & jnp.int32(-65536), jnp.float32)
    a3 = a_ref[...]
    bb = b_ref[...]
    q0 = jnp.dot(a3[:, 0, :], w2_ref[...],
                 preferred_element_type=jnp.float32) + bb
    q1 = jnp.dot(a3[:, 1, :], w2_ref[...],
                 preferred_element_type=jnp.float32) + bb
    o_ref[:, 0, :] = jnp.concatenate(
        [lo[:, :D_PACK], hi[:, :D_PACK]], axis=1) + q0
    o_ref[:, 1, :] = jnp.concatenate(
        [lo[:, D_PACK:], hi[:, D_PACK:]], axis=1) + q1


def _gather_add_body(p1_hbm, p3_hbm, ei_hbm, g_hbm,
                     idx_s, idx_d, rows_s, rows_d, rows_g,
                     sem_g0, sem_g1, sem_o0, sem_o1):
    sems_g = (sem_g0, sem_g1)
    sems_o = (sem_o0, sem_o1)
    wid = lax.axis_index("s") * NUM_CORES + lax.axis_index("c")

    pltpu.sync_copy(ei_hbm.at[0, wid], idx_s)
    pltpu.sync_copy(ei_hbm.at[1, wid], idx_d)

    def issue_gathers(i, b):
        off = i * CHUNK
        slot = lax.rem(i, NSLOT)
        pltpu.async_copy(
            p1_hbm.at[idx_s.at[pl.ds(off, CHUNK)]], rows_s.at[slot], sems_g[b])
        pltpu.async_copy(
            p3_hbm.at[idx_d.at[pl.ds(off, CHUNK)]], rows_d.at[slot], sems_g[b])

    def wait_gathers(i, b):
        off = i * CHUNK
        slot = lax.rem(i, NSLOT)
        pltpu.make_async_copy(
            p1_hbm.at[idx_s.at[pl.ds(off, CHUNK)]], rows_s.at[slot],
            sems_g[b]).wait()
        pltpu.make_async_copy(
            p3_hbm.at[idx_d.at[pl.ds(off, CHUNK)]], rows_d.at[slot],
            sems_g[b]).wait()

    def out_slice(i):
        return g_hbm.at[pl.ds((wid * E_PER_W + i * CHUNK) // 2, CHUNK // 2)]

    def process(i, b):
        wait_gathers(i, b)

        @pl.when(i + NBUF < NITER)
        def _():
            issue_gathers(i + NBUF, b)

        @pl.when(i >= NBUF)
        def _():
            pltpu.make_async_copy(rows_g.at[b], out_slice(i - NBUF),
                                  sems_o[b]).wait()

        slot = lax.rem(i, NSLOT)

        @plsc.parallel_loop(0, CHUNK // 2, unroll=2)
        def _add(r2):
            for half in range(2):
                r = r2 * 2 + half
                for c in range(D_PACK // 16):
                    sl = pl.ds(c * 16, 16)
                    s = plsc.bitcast(rows_s[slot, r, sl], jnp.bfloat16)
                    d = plsc.bitcast(rows_d[slot, r, sl], jnp.bfloat16)
                    rows_g[b, r2, pl.ds(half * D_PACK + c * 16, 16)] = (
                        plsc.bitcast(s + d, jnp.int32))

        pltpu.async_copy(rows_g.at[b], out_slice(i), sems_o[b])

    for b in range(NBUF):
        issue_gathers(jnp.int32(b), b)

    NMAIN = (NITER // NBUF) * NBUF

    @pl.loop(0, NMAIN, step=NBUF)
    def _outer(i0):
        for b in range(NBUF):
            process(i0 + b, b)

    for i in range(NMAIN, NITER):
        process(jnp.int32(i), i % NBUF)

    for b in range(NBUF):
        i = NITER - 1 - ((NITER - 1 - b) % NBUF)
        pltpu.make_async_copy(rows_g.at[b], out_slice(jnp.int32(i)),
                              sems_o[b]).wait()


@jax.jit
def kernel(x, edge_attr, edge_index, W, b):
    W1 = W[:D_FEAT]
    W2 = W[D_FEAT:D_FEAT + D_EDGE]
    W3 = W[D_FEAT + D_EDGE:]

    BN = 2000
    P1, P3 = pl.pallas_call(
        _proj_body,
        grid=(N_NODES // BN,),
        in_specs=[
            pl.BlockSpec((BN, D_FEAT), lambda i: (i, 0)),
            pl.BlockSpec((D_FEAT, D_OUT), lambda i: (0, 0)),
            pl.BlockSpec((D_FEAT, D_OUT), lambda i: (0, 0)),
        ],
        out_specs=[
            pl.BlockSpec((BN, D_PACK), lambda i: (i, 0)),
            pl.BlockSpec((BN, D_PACK), lambda i: (i, 0)),
        ],
        out_shape=[
            jax.ShapeDtypeStruct((N_NODES, D_PACK), jnp.int32),
            jax.ShapeDtypeStruct((N_NODES, D_PACK), jnp.int32),
        ],
    )(x, W1, W3)

    ei = edge_index.reshape(2, NW, E_PER_W)
    mesh = plsc.VectorSubcoreMesh(core_axis_name="c", subcore_axis_name="s")
    Gw = pl.kernel(
        _gather_add_body,
        out_type=jax.ShapeDtypeStruct((N_EDGES // 2, D_OUT), jnp.int32),
        mesh=mesh,
        compiler_params=pltpu.CompilerParams(
            use_tc_tiling_on_sc=False, needs_layout_passes=False),
        scratch_types=[
            pltpu.VMEM((E_PER_W,), jnp.int32),
            pltpu.VMEM((E_PER_W,), jnp.int32),
            pltpu.VMEM((NSLOT, CHUNK, D_PACK), jnp.int32),
            pltpu.VMEM((NSLOT, CHUNK, D_PACK), jnp.int32),
            pltpu.VMEM((NBUF, CHUNK // 2, D_OUT), jnp.int32),
            pltpu.SemaphoreType.DMA,
            pltpu.SemaphoreType.DMA,
            pltpu.SemaphoreType.DMA,
            pltpu.SemaphoreType.DMA,
        ],
    )(P1, P3, ei)

    BE = 16000
    BEH = BE // 2
    b2 = b.reshape(1, D_OUT)
    attr3 = edge_attr.reshape(N_EDGES // 2, 2, D_EDGE)
    out3 = pl.pallas_call(
        _final_body,
        grid=(N_EDGES // BE,),
        in_specs=[
            pl.BlockSpec((BEH, D_OUT), lambda i: (i, 0)),
            pl.BlockSpec((BEH, 2, D_EDGE), lambda i: (i, 0, 0)),
            pl.BlockSpec((D_EDGE, D_OUT), lambda i: (0, 0)),
            pl.BlockSpec((1, D_OUT), lambda i: (0, 0)),
        ],
        out_specs=pl.BlockSpec((BEH, 2, D_OUT), lambda i: (i, 0, 0)),
        out_shape=jax.ShapeDtypeStruct((N_EDGES // 2, 2, D_OUT), jnp.float32),
    )(Gw, attr3, W2, b2)
    return out3.reshape(N_EDGES, D_OUT)
